# baseline (speedup 1.0000x reference)
import jax
import jax.numpy as jnp
from jax.experimental import pallas as pl
from jax.experimental.pallas import tpu as pltpu


def _pos_kernel(row_ref, col_ref, out_ref, scratch, sem):
    h = row_ref.shape[0]
    w = col_ref.shape[0]
    f_half = row_ref.shape[1]
    b = out_ref.shape[0]
    half = h // 8
    copies = []
    for lo in range(0, h, half):
        scratch[lo:lo + half, :, 0:f_half] = jnp.broadcast_to(
            col_ref[...][None, :, :], (half, w, f_half)
        )
        scratch[lo:lo + half, :, f_half:2 * f_half] = jnp.broadcast_to(
            row_ref[lo:lo + half, :][:, None, :], (half, w, f_half)
        )
        for i in range(b):
            c = pltpu.make_async_copy(
                scratch.at[pl.ds(lo, half)],
                out_ref.at[i, pl.ds(lo, half)],
                sem,
            )
            c.start()
            copies.append(c)
    for c in copies:
        c.wait()


def kernel(mask, row_embed, col_embed):
    b, h, w = mask.shape
    f_half = row_embed.shape[1]
    f = 2 * f_half
    out = pl.pallas_call(
        _pos_kernel,
        out_specs=pl.BlockSpec(memory_space=pl.ANY),
        out_shape=jax.ShapeDtypeStruct((b, h, w, f), jnp.float32),
        scratch_shapes=[
            pltpu.VMEM((h, w, f), jnp.float32),
            pltpu.SemaphoreType.DMA,
        ],
    )(row_embed, col_embed)
    return jnp.transpose(out, (0, 3, 1, 2))


# speedup vs baseline: 1.0081x; 1.0081x over previous
import jax
import jax.numpy as jnp
from jax.experimental import pallas as pl
from jax.experimental.pallas import tpu as pltpu


def _pos_kernel(row_ref, col_ref, out_ref, scratch, sem):
    h = row_ref.shape[0]
    w = col_ref.shape[0]
    f_half = row_ref.shape[1]
    b = out_ref.shape[0]
    half = h // 4
    copies = []
    for lo in range(0, h, half):
        scratch[lo:lo + half, :, 0:f_half] = jnp.broadcast_to(
            col_ref[...][None, :, :], (half, w, f_half)
        )
        scratch[lo:lo + half, :, f_half:2 * f_half] = jnp.broadcast_to(
            row_ref[lo:lo + half, :][:, None, :], (half, w, f_half)
        )
        for i in range(b):
            c = pltpu.make_async_copy(
                scratch.at[pl.ds(lo, half)],
                out_ref.at[i, pl.ds(lo, half)],
                sem,
            )
            c.start()
            copies.append(c)
    for c in copies:
        c.wait()


def kernel(mask, row_embed, col_embed):
    b, h, w = mask.shape
    f_half = row_embed.shape[1]
    f = 2 * f_half
    out = pl.pallas_call(
        _pos_kernel,
        out_specs=pl.BlockSpec(memory_space=pl.ANY),
        out_shape=jax.ShapeDtypeStruct((b, h, w, f), jnp.float32),
        scratch_shapes=[
            pltpu.VMEM((h, w, f), jnp.float32),
            pltpu.SemaphoreType.DMA,
        ],
    )(row_embed, col_embed)
    return jnp.transpose(out, (0, 3, 1, 2))
